# Initial kernel scaffold; baseline (speedup 1.0000x reference)
import jax
import jax.numpy as jnp
from jax.experimental import pallas as pl


def kernel(xyz1, xyz2, features1, features2, W0, b0, g0, be0, W1, b1, g1, be1):
    raise NotImplementedError("write your pallas kernel here")


# speedup vs baseline: 17.0636x; 17.0636x over previous
import functools

import jax
import jax.numpy as jnp
from jax.experimental import pallas as pl

K_NN = 16


def _stage1_body(x1_ref, x2t_ref, f1_ref, f2_ref, w0_ref, p_ref,
                 h0_ref, stats_ref, *, cnt_ln):
    b = pl.program_id(0)
    nt = pl.program_id(1)

    x1 = x1_ref[0]
    x2t = x2t_ref[0]
    f1 = f1_ref[0]
    f2 = f2_ref[0]

    x1r = x1.astype(jnp.bfloat16).astype(jnp.float32)
    x2r = x2t.astype(jnp.bfloat16).astype(jnp.float32)
    g = (x2r[:, 0:1] * x1r[0:1, :] + x2r[:, 1:2] * x1r[1:2, :]
         + x2r[:, 2:3] * x1r[2:3, :])
    n2 = (x2t[:, 0:1] * x2t[:, 0:1] + x2t[:, 1:2] * x2t[:, 1:2]
          + x2t[:, 2:3] * x2t[:, 2:3])
    n1 = (x1[0:1, :] * x1[0:1, :] + x1[1:2, :] * x1[1:2, :]
          + x1[2:3, :] * x1[2:3, :])
    d = (-2.0 * g + n1) + n2

    t = jnp.full((1, d.shape[1]), -jnp.inf, dtype=jnp.float32)
    for _ in range(K_NN):
        dm = jnp.where(d > t, d, jnp.inf)
        t = jnp.min(dm, axis=0, keepdims=True)

    mask = d <= t
    r = jnp.where(mask, 1.0 / (d + 1e-8), 0.0)
    norm = jnp.sum(r, axis=0, keepdims=True)
    w = r / norm

    interp = jax.lax.dot_general(f2, w, (((1,), (0,)), ((), ())),
                                 preferred_element_type=jnp.float32,
                                 precision=jax.lax.Precision.HIGHEST)
    nf = jnp.concatenate([f1, interp], axis=0)
    h = jax.lax.dot_general(w0_ref[...].astype(jnp.bfloat16),
                            nf.astype(jnp.bfloat16), (((1,), (0,)), ((), ())),
                            preferred_element_type=jnp.float32)
    h = h + p_ref[:, 0:1]

    h0_ref[0] = h

    tn = h.shape[1]
    s = jnp.sum(h.reshape(h.shape[0], tn // cnt_ln, cnt_ln), axis=1)
    ss = jnp.sum((h * h).reshape(h.shape[0], tn // cnt_ln, cnt_ln), axis=1)

    @pl.when(jnp.logical_and(b == 0, nt == 0))
    def _():
        stats_ref[...] = jnp.zeros_like(stats_ref)

    stats_ref[0] += s
    stats_ref[1] += ss


def _stage2_body(h0_ref, stats_ref, w1_ref, p_ref, h1_ref, stats2_ref,
                 *, count, cnt_ln):
    b = pl.program_id(0)
    nt = pl.program_id(1)

    s = jnp.sum(stats_ref[0], axis=1, keepdims=True)
    ss = jnp.sum(stats_ref[1], axis=1, keepdims=True)
    mean = s / count
    var = ss / count - mean * mean
    rstd = jax.lax.rsqrt(var + 1e-5)

    h0 = h0_ref[0]
    a = (h0 - mean) * rstd * p_ref[:, 1:2] + p_ref[:, 2:3]
    a = jnp.maximum(a, 0.0)
    h1 = jax.lax.dot_general(w1_ref[...].astype(jnp.bfloat16),
                             a.astype(jnp.bfloat16), (((1,), (0,)), ((), ())),
                             preferred_element_type=jnp.float32)
    h1 = h1 + p_ref[:, 3:4]
    h1_ref[0] = h1

    tn = h1.shape[1]
    s2 = jnp.sum(h1.reshape(h1.shape[0], tn // cnt_ln, cnt_ln), axis=1)
    ss2 = jnp.sum((h1 * h1).reshape(h1.shape[0], tn // cnt_ln, cnt_ln), axis=1)

    @pl.when(jnp.logical_and(b == 0, nt == 0))
    def _():
        stats2_ref[...] = jnp.zeros_like(stats2_ref)

    stats2_ref[0] += s2
    stats2_ref[1] += ss2


def _stage3_body(h1_ref, stats2_ref, p_ref, out_ref, *, count):
    s = jnp.sum(stats2_ref[0], axis=1, keepdims=True)
    ss = jnp.sum(stats2_ref[1], axis=1, keepdims=True)
    mean = s / count
    var = ss / count - mean * mean
    rstd = jax.lax.rsqrt(var + 1e-5)

    h1 = h1_ref[0]
    y = (h1 - mean) * rstd * p_ref[:, 4:5] + p_ref[:, 5:6]
    out_ref[0] = jnp.maximum(y, 0.0)


def kernel(xyz1, xyz2, features1, features2, W0, b0, g0, be0, W1, b1, g1, be1):
    B, _, N = xyz1.shape
    M = xyz2.shape[2]
    C1 = features1.shape[1]
    C2 = features2.shape[1]
    Cout = W0.shape[0]
    f32 = jnp.float32

    TN = min(512, N)
    TN2 = min(1024, N)
    LN = 128

    x2t = jnp.transpose(xyz2, (0, 2, 1))
    params = jnp.stack([b0, g0, be0, b1, g1, be1,
                        jnp.zeros_like(b0), jnp.zeros_like(b0)], axis=1)

    count = float(B * N)

    h0, stats = pl.pallas_call(
        functools.partial(_stage1_body, cnt_ln=LN),
        grid=(B, N // TN),
        in_specs=[
            pl.BlockSpec((1, 3, TN), lambda b, n: (b, 0, n)),
            pl.BlockSpec((1, M, 3), lambda b, n: (b, 0, 0)),
            pl.BlockSpec((1, C1, TN), lambda b, n: (b, 0, n)),
            pl.BlockSpec((1, C2, M), lambda b, n: (b, 0, 0)),
            pl.BlockSpec((Cout, C1 + C2), lambda b, n: (0, 0)),
            pl.BlockSpec((Cout, 8), lambda b, n: (0, 0)),
        ],
        out_specs=[
            pl.BlockSpec((1, Cout, TN), lambda b, n: (b, 0, n)),
            pl.BlockSpec((2, Cout, LN), lambda b, n: (0, 0, 0)),
        ],
        out_shape=[
            jax.ShapeDtypeStruct((B, Cout, N), f32),
            jax.ShapeDtypeStruct((2, Cout, LN), f32),
        ],
    )(xyz1, x2t, features1, features2, W0, params)

    h1, stats2 = pl.pallas_call(
        functools.partial(_stage2_body, count=count, cnt_ln=LN),
        grid=(B, N // TN2),
        in_specs=[
            pl.BlockSpec((1, Cout, TN2), lambda b, n: (b, 0, n)),
            pl.BlockSpec((2, Cout, LN), lambda b, n: (0, 0, 0)),
            pl.BlockSpec((Cout, Cout), lambda b, n: (0, 0)),
            pl.BlockSpec((Cout, 8), lambda b, n: (0, 0)),
        ],
        out_specs=[
            pl.BlockSpec((1, Cout, TN2), lambda b, n: (b, 0, n)),
            pl.BlockSpec((2, Cout, LN), lambda b, n: (0, 0, 0)),
        ],
        out_shape=[
            jax.ShapeDtypeStruct((B, Cout, N), f32),
            jax.ShapeDtypeStruct((2, Cout, LN), f32),
        ],
    )(h0, stats, W1, params)

    out = pl.pallas_call(
        functools.partial(_stage3_body, count=count),
        grid=(B, N // TN2),
        in_specs=[
            pl.BlockSpec((1, Cout, TN2), lambda b, n: (b, 0, n)),
            pl.BlockSpec((2, Cout, LN), lambda b, n: (0, 0, 0)),
            pl.BlockSpec((Cout, 8), lambda b, n: (0, 0)),
        ],
        out_specs=pl.BlockSpec((1, Cout, TN2), lambda b, n: (b, 0, n)),
        out_shape=jax.ShapeDtypeStruct((B, Cout, N), f32),
    )(h1, stats2, params)

    return out


# speedup vs baseline: 22.0196x; 1.2904x over previous
import functools

import jax
import jax.numpy as jnp
from jax.experimental import pallas as pl

K_NN = 16


def _stage1_body(x1_ref, x2t_ref, f1_ref, f2_ref, w0_ref, p_ref,
                 h0_ref, stats_ref, *, cnt_ln):
    b = pl.program_id(0)
    nt = pl.program_id(1)

    x1 = x1_ref[0]
    x2t = x2t_ref[0]
    f1 = f1_ref[0]
    f2 = f2_ref[0]

    x1r = x1.astype(jnp.bfloat16).astype(jnp.float32)
    x2r = x2t.astype(jnp.bfloat16).astype(jnp.float32)
    g = (x2r[:, 0:1] * x1r[0:1, :] + x2r[:, 1:2] * x1r[1:2, :]
         + x2r[:, 2:3] * x1r[2:3, :])
    n2 = (x2t[:, 0:1] * x2t[:, 0:1] + x2t[:, 1:2] * x2t[:, 1:2]
          + x2t[:, 2:3] * x2t[:, 2:3])
    n1 = (x1[0:1, :] * x1[0:1, :] + x1[1:2, :] * x1[1:2, :]
          + x1[2:3, :] * x1[2:3, :])
    d = (-2.0 * g + n1) + n2

    nblk = 8
    bs = d.shape[0] // nblk
    S = [d[i * bs:(i + 1) * bs, :] for i in range(nblk)]
    net = [(0, 1), (2, 3), (4, 5), (6, 7),
           (0, 2), (1, 3), (4, 6), (5, 7),
           (1, 2), (5, 6),
           (0, 4), (1, 5), (2, 6), (3, 7),
           (2, 4), (3, 5),
           (1, 2), (3, 4), (5, 6)]
    for i, j in net:
        lo = jnp.minimum(S[i], S[j])
        hi = jnp.maximum(S[i], S[j])
        S[i], S[j] = lo, hi
    t = None
    for it in range(K_NN):
        t = jnp.min(S[0], axis=0, keepdims=True)
        if it < K_NN - 1:
            pop = S[0] == t
            for i in range(nblk - 1):
                S[i] = jnp.where(pop, S[i + 1], S[i])
            S[nblk - 1] = jnp.where(pop, jnp.inf, S[nblk - 1])

    mask = d <= t
    r = jnp.where(mask, 1.0 / (d + 1e-8), 0.0)
    norm = jnp.sum(r, axis=0, keepdims=True)
    w = r / norm

    interp = jax.lax.dot_general(f2, w, (((1,), (0,)), ((), ())),
                                 preferred_element_type=jnp.float32,
                                 precision=jax.lax.Precision.HIGHEST)
    nf = jnp.concatenate([f1, interp], axis=0)
    h = jax.lax.dot_general(w0_ref[...].astype(jnp.bfloat16),
                            nf.astype(jnp.bfloat16), (((1,), (0,)), ((), ())),
                            preferred_element_type=jnp.float32)
    h = h + p_ref[:, 0:1]

    h0_ref[0] = h

    tn = h.shape[1]
    hh = h * h
    s = h[:, 0:cnt_ln]
    ss = hh[:, 0:cnt_ln]
    for q in range(1, tn // cnt_ln):
        s = s + h[:, q * cnt_ln:(q + 1) * cnt_ln]
        ss = ss + hh[:, q * cnt_ln:(q + 1) * cnt_ln]

    @pl.when(jnp.logical_and(b == 0, nt == 0))
    def _():
        stats_ref[...] = jnp.zeros_like(stats_ref)

    stats_ref[0] += s
    stats_ref[1] += ss


def _stage2_body(h0_ref, stats_ref, w1_ref, p_ref, h1_ref, stats2_ref,
                 *, count, cnt_ln):
    b = pl.program_id(0)
    nt = pl.program_id(1)

    s = jnp.sum(stats_ref[0], axis=1, keepdims=True)
    ss = jnp.sum(stats_ref[1], axis=1, keepdims=True)
    mean = s / count
    var = ss / count - mean * mean
    rstd = jax.lax.rsqrt(var + 1e-5)

    h0 = h0_ref[0]
    a = (h0 - mean) * rstd * p_ref[:, 1:2] + p_ref[:, 2:3]
    a = jnp.maximum(a, 0.0)
    h1 = jax.lax.dot_general(w1_ref[...].astype(jnp.bfloat16),
                             a.astype(jnp.bfloat16), (((1,), (0,)), ((), ())),
                             preferred_element_type=jnp.float32)
    h1 = h1 + p_ref[:, 3:4]
    h1_ref[0] = h1

    tn = h1.shape[1]
    hh1 = h1 * h1
    s2 = h1[:, 0:cnt_ln]
    ss2 = hh1[:, 0:cnt_ln]
    for q in range(1, tn // cnt_ln):
        s2 = s2 + h1[:, q * cnt_ln:(q + 1) * cnt_ln]
        ss2 = ss2 + hh1[:, q * cnt_ln:(q + 1) * cnt_ln]

    @pl.when(jnp.logical_and(b == 0, nt == 0))
    def _():
        stats2_ref[...] = jnp.zeros_like(stats2_ref)

    stats2_ref[0] += s2
    stats2_ref[1] += ss2


def _stage3_body(h1_ref, stats2_ref, p_ref, out_ref, *, count):
    s = jnp.sum(stats2_ref[0], axis=1, keepdims=True)
    ss = jnp.sum(stats2_ref[1], axis=1, keepdims=True)
    mean = s / count
    var = ss / count - mean * mean
    rstd = jax.lax.rsqrt(var + 1e-5)

    h1 = h1_ref[0]
    y = (h1 - mean) * rstd * p_ref[:, 4:5] + p_ref[:, 5:6]
    out_ref[0] = jnp.maximum(y, 0.0)


def kernel(xyz1, xyz2, features1, features2, W0, b0, g0, be0, W1, b1, g1, be1):
    B, _, N = xyz1.shape
    M = xyz2.shape[2]
    C1 = features1.shape[1]
    C2 = features2.shape[1]
    Cout = W0.shape[0]
    f32 = jnp.float32

    TN = min(512, N)
    TN2 = min(1024, N)
    LN = 128

    x2t = jnp.transpose(xyz2, (0, 2, 1))
    params = jnp.stack([b0, g0, be0, b1, g1, be1,
                        jnp.zeros_like(b0), jnp.zeros_like(b0)], axis=1)

    count = float(B * N)

    h0, stats = pl.pallas_call(
        functools.partial(_stage1_body, cnt_ln=LN),
        grid=(B, N // TN),
        in_specs=[
            pl.BlockSpec((1, 3, TN), lambda b, n: (b, 0, n)),
            pl.BlockSpec((1, M, 3), lambda b, n: (b, 0, 0)),
            pl.BlockSpec((1, C1, TN), lambda b, n: (b, 0, n)),
            pl.BlockSpec((1, C2, M), lambda b, n: (b, 0, 0)),
            pl.BlockSpec((Cout, C1 + C2), lambda b, n: (0, 0)),
            pl.BlockSpec((Cout, 8), lambda b, n: (0, 0)),
        ],
        out_specs=[
            pl.BlockSpec((1, Cout, TN), lambda b, n: (b, 0, n)),
            pl.BlockSpec((2, Cout, LN), lambda b, n: (0, 0, 0)),
        ],
        out_shape=[
            jax.ShapeDtypeStruct((B, Cout, N), f32),
            jax.ShapeDtypeStruct((2, Cout, LN), f32),
        ],
    )(xyz1, x2t, features1, features2, W0, params)

    h1, stats2 = pl.pallas_call(
        functools.partial(_stage2_body, count=count, cnt_ln=LN),
        grid=(B, N // TN2),
        in_specs=[
            pl.BlockSpec((1, Cout, TN2), lambda b, n: (b, 0, n)),
            pl.BlockSpec((2, Cout, LN), lambda b, n: (0, 0, 0)),
            pl.BlockSpec((Cout, Cout), lambda b, n: (0, 0)),
            pl.BlockSpec((Cout, 8), lambda b, n: (0, 0)),
        ],
        out_specs=[
            pl.BlockSpec((1, Cout, TN2), lambda b, n: (b, 0, n)),
            pl.BlockSpec((2, Cout, LN), lambda b, n: (0, 0, 0)),
        ],
        out_shape=[
            jax.ShapeDtypeStruct((B, Cout, N), f32),
            jax.ShapeDtypeStruct((2, Cout, LN), f32),
        ],
    )(h0, stats, W1, params)

    out = pl.pallas_call(
        functools.partial(_stage3_body, count=count),
        grid=(B, N // TN2),
        in_specs=[
            pl.BlockSpec((1, Cout, TN2), lambda b, n: (b, 0, n)),
            pl.BlockSpec((2, Cout, LN), lambda b, n: (0, 0, 0)),
            pl.BlockSpec((Cout, 8), lambda b, n: (0, 0)),
        ],
        out_specs=pl.BlockSpec((1, Cout, TN2), lambda b, n: (b, 0, n)),
        out_shape=jax.ShapeDtypeStruct((B, Cout, N), f32),
    )(h1, stats2, params)

    return out


# speedup vs baseline: 24.8886x; 1.1303x over previous
import functools

import jax
import jax.numpy as jnp
from jax.experimental import pallas as pl

K_NN = 16


def _stage1_body(x1_ref, x2t_ref, f1_ref, f2_ref, w0_ref, p_ref,
                 h0_ref, stats_ref, *, cnt_ln):
    b = pl.program_id(0)
    nt = pl.program_id(1)

    x1 = x1_ref[0]
    x2t = x2t_ref[0]
    f1 = f1_ref[0]
    f2 = f2_ref[0]

    x1r = x1.astype(jnp.bfloat16).astype(jnp.float32)
    x2r = x2t.astype(jnp.bfloat16).astype(jnp.float32)
    g = (x2r[:, 0:1] * x1r[0:1, :] + x2r[:, 1:2] * x1r[1:2, :]
         + x2r[:, 2:3] * x1r[2:3, :])
    n2 = (x2t[:, 0:1] * x2t[:, 0:1] + x2t[:, 1:2] * x2t[:, 1:2]
          + x2t[:, 2:3] * x2t[:, 2:3])
    n1 = (x1[0:1, :] * x1[0:1, :] + x1[1:2, :] * x1[1:2, :]
          + x1[2:3, :] * x1[2:3, :])
    d = (-2.0 * g + n1) + n2

    nblk = 8
    bs = d.shape[0] // nblk
    S = [d[i * bs:(i + 1) * bs, :] for i in range(nblk)]
    net = [(0, 1), (2, 3), (4, 5), (6, 7),
           (0, 2), (1, 3), (4, 6), (5, 7),
           (1, 2), (5, 6),
           (0, 4), (1, 5), (2, 6), (3, 7),
           (2, 4), (3, 5),
           (1, 2), (3, 4), (5, 6)]
    for i, j in net:
        lo = jnp.minimum(S[i], S[j])
        hi = jnp.maximum(S[i], S[j])
        S[i], S[j] = lo, hi
    t = None
    for it in range(K_NN):
        t = jnp.min(S[0], axis=0, keepdims=True)
        if it < K_NN - 1:
            pop = S[0] == t
            for i in range(nblk - 1):
                S[i] = jnp.where(pop, S[i + 1], S[i])
            S[nblk - 1] = jnp.where(pop, jnp.inf, S[nblk - 1])

    mask = d <= t
    r = jnp.where(mask, 1.0 / (d + 1e-8), 0.0)
    norm = jnp.sum(r, axis=0, keepdims=True)
    w = r / norm

    interp = jax.lax.dot_general(f2.astype(jnp.bfloat16),
                                 w.astype(jnp.bfloat16),
                                 (((1,), (0,)), ((), ())),
                                 preferred_element_type=jnp.float32)
    nf = jnp.concatenate([f1, interp], axis=0)
    h = jax.lax.dot_general(w0_ref[...].astype(jnp.bfloat16),
                            nf.astype(jnp.bfloat16), (((1,), (0,)), ((), ())),
                            preferred_element_type=jnp.float32)
    h = h + p_ref[:, 0:1]

    h0_ref[0] = h

    tn = h.shape[1]
    hh = h * h
    s = h[:, 0:cnt_ln]
    ss = hh[:, 0:cnt_ln]
    for q in range(1, tn // cnt_ln):
        s = s + h[:, q * cnt_ln:(q + 1) * cnt_ln]
        ss = ss + hh[:, q * cnt_ln:(q + 1) * cnt_ln]

    @pl.when(jnp.logical_and(b == 0, nt == 0))
    def _():
        stats_ref[...] = jnp.zeros_like(stats_ref)

    stats_ref[0] += s
    stats_ref[1] += ss


def _stage2_body(h0_ref, stats_ref, w1_ref, p_ref, h1_ref, stats2_ref,
                 *, count, cnt_ln):
    b = pl.program_id(0)
    nt = pl.program_id(1)

    s = jnp.sum(stats_ref[0], axis=1, keepdims=True)
    ss = jnp.sum(stats_ref[1], axis=1, keepdims=True)
    mean = s / count
    var = ss / count - mean * mean
    rstd = jax.lax.rsqrt(var + 1e-5)

    h0 = h0_ref[0]
    a = (h0 - mean) * rstd * p_ref[:, 1:2] + p_ref[:, 2:3]
    a = jnp.maximum(a, 0.0)
    h1 = jax.lax.dot_general(w1_ref[...].astype(jnp.bfloat16),
                             a.astype(jnp.bfloat16), (((1,), (0,)), ((), ())),
                             preferred_element_type=jnp.float32)
    h1 = h1 + p_ref[:, 3:4]
    h1_ref[0] = h1

    tn = h1.shape[1]
    hh1 = h1 * h1
    s2 = h1[:, 0:cnt_ln]
    ss2 = hh1[:, 0:cnt_ln]
    for q in range(1, tn // cnt_ln):
        s2 = s2 + h1[:, q * cnt_ln:(q + 1) * cnt_ln]
        ss2 = ss2 + hh1[:, q * cnt_ln:(q + 1) * cnt_ln]

    @pl.when(jnp.logical_and(b == 0, nt == 0))
    def _():
        stats2_ref[...] = jnp.zeros_like(stats2_ref)

    stats2_ref[0] += s2
    stats2_ref[1] += ss2


def _stage3_body(h1_ref, stats2_ref, p_ref, out_ref, *, count):
    s = jnp.sum(stats2_ref[0], axis=1, keepdims=True)
    ss = jnp.sum(stats2_ref[1], axis=1, keepdims=True)
    mean = s / count
    var = ss / count - mean * mean
    rstd = jax.lax.rsqrt(var + 1e-5)

    h1 = h1_ref[0]
    y = (h1 - mean) * rstd * p_ref[:, 4:5] + p_ref[:, 5:6]
    out_ref[0] = jnp.maximum(y, 0.0)


def kernel(xyz1, xyz2, features1, features2, W0, b0, g0, be0, W1, b1, g1, be1):
    B, _, N = xyz1.shape
    M = xyz2.shape[2]
    C1 = features1.shape[1]
    C2 = features2.shape[1]
    Cout = W0.shape[0]
    f32 = jnp.float32

    TN = min(512, N)
    TN2 = min(1024, N)
    LN = 128

    x2t = jnp.transpose(xyz2, (0, 2, 1))
    params = jnp.stack([b0, g0, be0, b1, g1, be1,
                        jnp.zeros_like(b0), jnp.zeros_like(b0)], axis=1)

    count = float(B * N)

    h0, stats = pl.pallas_call(
        functools.partial(_stage1_body, cnt_ln=LN),
        grid=(B, N // TN),
        in_specs=[
            pl.BlockSpec((1, 3, TN), lambda b, n: (b, 0, n)),
            pl.BlockSpec((1, M, 3), lambda b, n: (b, 0, 0)),
            pl.BlockSpec((1, C1, TN), lambda b, n: (b, 0, n)),
            pl.BlockSpec((1, C2, M), lambda b, n: (b, 0, 0)),
            pl.BlockSpec((Cout, C1 + C2), lambda b, n: (0, 0)),
            pl.BlockSpec((Cout, 8), lambda b, n: (0, 0)),
        ],
        out_specs=[
            pl.BlockSpec((1, Cout, TN), lambda b, n: (b, 0, n)),
            pl.BlockSpec((2, Cout, LN), lambda b, n: (0, 0, 0)),
        ],
        out_shape=[
            jax.ShapeDtypeStruct((B, Cout, N), f32),
            jax.ShapeDtypeStruct((2, Cout, LN), f32),
        ],
    )(xyz1, x2t, features1, features2, W0, params)

    h1, stats2 = pl.pallas_call(
        functools.partial(_stage2_body, count=count, cnt_ln=LN),
        grid=(B, N // TN2),
        in_specs=[
            pl.BlockSpec((1, Cout, TN2), lambda b, n: (b, 0, n)),
            pl.BlockSpec((2, Cout, LN), lambda b, n: (0, 0, 0)),
            pl.BlockSpec((Cout, Cout), lambda b, n: (0, 0)),
            pl.BlockSpec((Cout, 8), lambda b, n: (0, 0)),
        ],
        out_specs=[
            pl.BlockSpec((1, Cout, TN2), lambda b, n: (b, 0, n)),
            pl.BlockSpec((2, Cout, LN), lambda b, n: (0, 0, 0)),
        ],
        out_shape=[
            jax.ShapeDtypeStruct((B, Cout, N), f32),
            jax.ShapeDtypeStruct((2, Cout, LN), f32),
        ],
    )(h0, stats, W1, params)

    out = pl.pallas_call(
        functools.partial(_stage3_body, count=count),
        grid=(B, N // TN2),
        in_specs=[
            pl.BlockSpec((1, Cout, TN2), lambda b, n: (b, 0, n)),
            pl.BlockSpec((2, Cout, LN), lambda b, n: (0, 0, 0)),
            pl.BlockSpec((Cout, 8), lambda b, n: (0, 0)),
        ],
        out_specs=pl.BlockSpec((1, Cout, TN2), lambda b, n: (b, 0, n)),
        out_shape=jax.ShapeDtypeStruct((B, Cout, N), f32),
    )(h1, stats2, params)

    return out


# speedup vs baseline: 31.7733x; 1.2766x over previous
import functools

import jax
import jax.numpy as jnp
from jax.experimental import pallas as pl
from jax.experimental.pallas import tpu as pltpu

K_NN = 16


def _stage1_body(x1_ref, x2t_ref, f1_ref, f2_ref, w0_ref, p_ref,
                 h0_ref, stats_ref, n2_ref, *, cnt_ln):
    b = pl.program_id(0)
    nt = pl.program_id(1)

    x1 = x1_ref[0]
    x2t = x2t_ref[0]
    f1 = f1_ref[0]
    f2 = f2_ref[0]

    @pl.when(nt == 0)
    def _():
        n2c = (x2t[:, 0:1] * x2t[:, 0:1] + x2t[:, 1:2] * x2t[:, 1:2]
               + x2t[:, 2:3] * x2t[:, 2:3])
        n2_ref[:, 0:1] = n2c

    g = jax.lax.dot_general(x2t.astype(jnp.bfloat16),
                            x1.astype(jnp.bfloat16),
                            (((1,), (0,)), ((), ())),
                            preferred_element_type=jnp.float32)
    n2 = n2_ref[:, 0:1]
    n1 = (x1[0:1, :] * x1[0:1, :] + x1[1:2, :] * x1[1:2, :]
          + x1[2:3, :] * x1[2:3, :])
    d = (-2.0 * g + n1) + n2

    nblk = 8
    bs = d.shape[0] // nblk
    S = [d[i * bs:(i + 1) * bs, :] for i in range(nblk)]
    net = [(0, 1), (2, 3), (4, 5), (6, 7),
           (0, 2), (1, 3), (4, 6), (5, 7),
           (1, 2), (5, 6),
           (0, 4), (1, 5), (2, 6), (3, 7),
           (2, 4), (3, 5),
           (1, 2), (3, 4), (5, 6)]
    for i, j in net:
        lo = jnp.minimum(S[i], S[j])
        hi = jnp.maximum(S[i], S[j])
        S[i], S[j] = lo, hi
    t = None
    for it in range(K_NN):
        t = jnp.min(S[0], axis=0, keepdims=True)
        if it < K_NN - 1:
            pop = S[0] == t
            depth = min(nblk - 1, K_NN - 1 - it)
            for i in range(depth):
                S[i] = jnp.where(pop, S[i + 1], S[i])
            if depth == nblk - 1:
                S[nblk - 1] = jnp.where(pop, jnp.inf, S[nblk - 1])

    mask = d <= t
    r = jnp.where(mask, 1.0 / (d + 1e-8), 0.0)
    norm = jnp.sum(r, axis=0, keepdims=True)
    w = r / norm

    interp = jax.lax.dot_general(f2.astype(jnp.bfloat16),
                                 w.astype(jnp.bfloat16),
                                 (((1,), (0,)), ((), ())),
                                 preferred_element_type=jnp.float32)
    nf = jnp.concatenate([f1, interp], axis=0)
    h = jax.lax.dot_general(w0_ref[...].astype(jnp.bfloat16),
                            nf.astype(jnp.bfloat16), (((1,), (0,)), ((), ())),
                            preferred_element_type=jnp.float32)
    h = h + p_ref[:, 0:1]

    h0_ref[0] = h

    tn = h.shape[1]
    hh = h * h
    s = h[:, 0:cnt_ln]
    ss = hh[:, 0:cnt_ln]
    for q in range(1, tn // cnt_ln):
        s = s + h[:, q * cnt_ln:(q + 1) * cnt_ln]
        ss = ss + hh[:, q * cnt_ln:(q + 1) * cnt_ln]

    @pl.when(jnp.logical_and(b == 0, nt == 0))
    def _():
        stats_ref[...] = jnp.zeros_like(stats_ref)

    stats_ref[0] += s
    stats_ref[1] += ss


def _stage2_body(h0_ref, stats_ref, w1_ref, p_ref, h1_ref, stats2_ref,
                 *, count, cnt_ln):
    b = pl.program_id(0)
    nt = pl.program_id(1)

    s = jnp.sum(stats_ref[0], axis=1, keepdims=True)
    ss = jnp.sum(stats_ref[1], axis=1, keepdims=True)
    mean = s / count
    var = ss / count - mean * mean
    rstd = jax.lax.rsqrt(var + 1e-5)

    h0 = h0_ref[0]
    a = (h0 - mean) * rstd * p_ref[:, 1:2] + p_ref[:, 2:3]
    a = jnp.maximum(a, 0.0)
    h1 = jax.lax.dot_general(w1_ref[...].astype(jnp.bfloat16),
                             a.astype(jnp.bfloat16), (((1,), (0,)), ((), ())),
                             preferred_element_type=jnp.float32)
    h1 = h1 + p_ref[:, 3:4]
    h1_ref[0] = h1

    tn = h1.shape[1]
    hh1 = h1 * h1
    s2 = h1[:, 0:cnt_ln]
    ss2 = hh1[:, 0:cnt_ln]
    for q in range(1, tn // cnt_ln):
        s2 = s2 + h1[:, q * cnt_ln:(q + 1) * cnt_ln]
        ss2 = ss2 + hh1[:, q * cnt_ln:(q + 1) * cnt_ln]

    @pl.when(jnp.logical_and(b == 0, nt == 0))
    def _():
        stats2_ref[...] = jnp.zeros_like(stats2_ref)

    stats2_ref[0] += s2
    stats2_ref[1] += ss2


def _stage3_body(h1_ref, stats2_ref, p_ref, out_ref, *, count):
    s = jnp.sum(stats2_ref[0], axis=1, keepdims=True)
    ss = jnp.sum(stats2_ref[1], axis=1, keepdims=True)
    mean = s / count
    var = ss / count - mean * mean
    rstd = jax.lax.rsqrt(var + 1e-5)

    h1 = h1_ref[0]
    y = (h1 - mean) * rstd * p_ref[:, 4:5] + p_ref[:, 5:6]
    out_ref[0] = jnp.maximum(y, 0.0)


def kernel(xyz1, xyz2, features1, features2, W0, b0, g0, be0, W1, b1, g1, be1):
    B, _, N = xyz1.shape
    M = xyz2.shape[2]
    C1 = features1.shape[1]
    C2 = features2.shape[1]
    Cout = W0.shape[0]
    f32 = jnp.float32

    TN = min(512, N)
    TN2 = min(1024, N)
    LN = 128

    x2t = jnp.transpose(xyz2, (0, 2, 1))
    params = jnp.stack([b0, g0, be0, b1, g1, be1,
                        jnp.zeros_like(b0), jnp.zeros_like(b0)], axis=1)

    count = float(B * N)

    h0, stats = pl.pallas_call(
        functools.partial(_stage1_body, cnt_ln=LN),
        grid=(B, N // TN),
        in_specs=[
            pl.BlockSpec((1, 3, TN), lambda b, n: (b, 0, n)),
            pl.BlockSpec((1, M, 3), lambda b, n: (b, 0, 0)),
            pl.BlockSpec((1, C1, TN), lambda b, n: (b, 0, n)),
            pl.BlockSpec((1, C2, M), lambda b, n: (b, 0, 0)),
            pl.BlockSpec((Cout, C1 + C2), lambda b, n: (0, 0)),
            pl.BlockSpec((Cout, 8), lambda b, n: (0, 0)),
        ],
        out_specs=[
            pl.BlockSpec((1, Cout, TN), lambda b, n: (b, 0, n)),
            pl.BlockSpec((2, Cout, LN), lambda b, n: (0, 0, 0)),
        ],
        out_shape=[
            jax.ShapeDtypeStruct((B, Cout, N), f32),
            jax.ShapeDtypeStruct((2, Cout, LN), f32),
        ],
        scratch_shapes=[pltpu.VMEM((M, 128), f32)],
    )(xyz1, x2t, features1, features2, W0, params)

    h1, stats2 = pl.pallas_call(
        functools.partial(_stage2_body, count=count, cnt_ln=LN),
        grid=(B, N // TN2),
        in_specs=[
            pl.BlockSpec((1, Cout, TN2), lambda b, n: (b, 0, n)),
            pl.BlockSpec((2, Cout, LN), lambda b, n: (0, 0, 0)),
            pl.BlockSpec((Cout, Cout), lambda b, n: (0, 0)),
            pl.BlockSpec((Cout, 8), lambda b, n: (0, 0)),
        ],
        out_specs=[
            pl.BlockSpec((1, Cout, TN2), lambda b, n: (b, 0, n)),
            pl.BlockSpec((2, Cout, LN), lambda b, n: (0, 0, 0)),
        ],
        out_shape=[
            jax.ShapeDtypeStruct((B, Cout, N), f32),
            jax.ShapeDtypeStruct((2, Cout, LN), f32),
        ],
    )(h0, stats, W1, params)

    out = pl.pallas_call(
        functools.partial(_stage3_body, count=count),
        grid=(B, N // TN2),
        in_specs=[
            pl.BlockSpec((1, Cout, TN2), lambda b, n: (b, 0, n)),
            pl.BlockSpec((2, Cout, LN), lambda b, n: (0, 0, 0)),
            pl.BlockSpec((Cout, 8), lambda b, n: (0, 0)),
        ],
        out_specs=pl.BlockSpec((1, Cout, TN2), lambda b, n: (b, 0, n)),
        out_shape=jax.ShapeDtypeStruct((B, Cout, N), f32),
    )(h1, stats2, params)

    return out


# speedup vs baseline: 33.9053x; 1.0671x over previous
import functools

import jax
import jax.numpy as jnp
from jax.experimental import pallas as pl
from jax.experimental.pallas import tpu as pltpu

K_NN = 16


def _stage1_body(x1_ref, x2t_ref, f1_ref, f2_ref, w0_ref, p_ref,
                 h0_ref, stats_ref, n2_ref, *, cnt_ln):
    b = pl.program_id(0)
    nt = pl.program_id(1)

    x1 = x1_ref[0]
    x2t = x2t_ref[0]
    f1 = f1_ref[0]
    f2 = f2_ref[0]

    @pl.when(nt == 0)
    def _():
        n2c = (x2t[:, 0:1] * x2t[:, 0:1] + x2t[:, 1:2] * x2t[:, 1:2]
               + x2t[:, 2:3] * x2t[:, 2:3])
        n2_ref[:, 0:1] = n2c

    g = jax.lax.dot_general(x2t.astype(jnp.bfloat16),
                            x1.astype(jnp.bfloat16),
                            (((1,), (0,)), ((), ())),
                            preferred_element_type=jnp.float32)
    n2 = n2_ref[:, 0:1]
    n1 = (x1[0:1, :] * x1[0:1, :] + x1[1:2, :] * x1[1:2, :]
          + x1[2:3, :] * x1[2:3, :])
    d = (-2.0 * g + n1) + n2

    nblk = 8
    bs = d.shape[0] // nblk
    S = [d[i * bs:(i + 1) * bs, :] for i in range(nblk)]
    net = [(0, 1), (2, 3), (4, 5), (6, 7),
           (0, 2), (1, 3), (4, 6), (5, 7),
           (1, 2), (5, 6),
           (0, 4), (1, 5), (2, 6), (3, 7),
           (2, 4), (3, 5),
           (1, 2), (3, 4), (5, 6)]
    for i, j in net:
        lo = jnp.minimum(S[i], S[j])
        hi = jnp.maximum(S[i], S[j])
        S[i], S[j] = lo, hi
    t = None
    for it in range(K_NN):
        t = jnp.min(S[0], axis=0, keepdims=True)
        if it < K_NN - 1:
            pop = S[0] == t
            depth = min(nblk - 1, K_NN - 1 - it)
            for i in range(depth):
                S[i] = jnp.where(pop, S[i + 1], S[i])
            if depth == nblk - 1:
                S[nblk - 1] = jnp.where(pop, jnp.inf, S[nblk - 1])

    mask = d <= t
    r = jnp.where(mask, 1.0 / (d + 1e-8), 0.0)
    norm = jnp.sum(r, axis=0, keepdims=True)
    w = r / norm

    interp = jax.lax.dot_general(f2.astype(jnp.bfloat16),
                                 w.astype(jnp.bfloat16),
                                 (((1,), (0,)), ((), ())),
                                 preferred_element_type=jnp.float32)
    nf = jnp.concatenate([f1, interp], axis=0)
    h = jax.lax.dot_general(w0_ref[...].astype(jnp.bfloat16),
                            nf.astype(jnp.bfloat16), (((1,), (0,)), ((), ())),
                            preferred_element_type=jnp.float32)
    h = h + p_ref[:, 0:1]

    h0_ref[0] = h

    tn = h.shape[1]
    hh = h * h
    s = h[:, 0:cnt_ln]
    ss = hh[:, 0:cnt_ln]
    for q in range(1, tn // cnt_ln):
        s = s + h[:, q * cnt_ln:(q + 1) * cnt_ln]
        ss = ss + hh[:, q * cnt_ln:(q + 1) * cnt_ln]

    @pl.when(jnp.logical_and(b == 0, nt == 0))
    def _():
        stats_ref[...] = jnp.zeros_like(stats_ref)

    stats_ref[0] += s
    stats_ref[1] += ss


def _stage2_body(h0_ref, stats_ref, w1_ref, p_ref, h1_ref, stats2_ref,
                 *, count, cnt_ln):
    b = pl.program_id(0)
    nt = pl.program_id(1)

    s = jnp.sum(stats_ref[0], axis=1, keepdims=True)
    ss = jnp.sum(stats_ref[1], axis=1, keepdims=True)
    mean = s / count
    var = ss / count - mean * mean
    rstd = jax.lax.rsqrt(var + 1e-5)

    h0 = h0_ref[0]
    a = (h0 - mean) * rstd * p_ref[:, 1:2] + p_ref[:, 2:3]
    a = jnp.maximum(a, 0.0)
    h1 = jax.lax.dot_general(w1_ref[...].astype(jnp.bfloat16),
                             a.astype(jnp.bfloat16), (((1,), (0,)), ((), ())),
                             preferred_element_type=jnp.float32)
    h1 = h1 + p_ref[:, 3:4]
    h1_ref[0] = h1

    tn = h1.shape[1]
    hh1 = h1 * h1
    s2 = h1[:, 0:cnt_ln]
    ss2 = hh1[:, 0:cnt_ln]
    for q in range(1, tn // cnt_ln):
        s2 = s2 + h1[:, q * cnt_ln:(q + 1) * cnt_ln]
        ss2 = ss2 + hh1[:, q * cnt_ln:(q + 1) * cnt_ln]

    @pl.when(jnp.logical_and(b == 0, nt == 0))
    def _():
        stats2_ref[...] = jnp.zeros_like(stats2_ref)

    stats2_ref[0] += s2
    stats2_ref[1] += ss2


def _stage3_body(h1_ref, stats2_ref, p_ref, out_ref, *, count):
    s = jnp.sum(stats2_ref[0], axis=1, keepdims=True)
    ss = jnp.sum(stats2_ref[1], axis=1, keepdims=True)
    mean = s / count
    var = ss / count - mean * mean
    rstd = jax.lax.rsqrt(var + 1e-5)

    h1 = h1_ref[0]
    y = (h1 - mean) * rstd * p_ref[:, 4:5] + p_ref[:, 5:6]
    out_ref[0] = jnp.maximum(y, 0.0)


def kernel(xyz1, xyz2, features1, features2, W0, b0, g0, be0, W1, b1, g1, be1):
    B, _, N = xyz1.shape
    M = xyz2.shape[2]
    C1 = features1.shape[1]
    C2 = features2.shape[1]
    Cout = W0.shape[0]
    f32 = jnp.float32

    TN = min(1024, N)
    TN2 = min(1024, N)
    LN = 128

    x2t = jnp.transpose(xyz2, (0, 2, 1))
    params = jnp.stack([b0, g0, be0, b1, g1, be1,
                        jnp.zeros_like(b0), jnp.zeros_like(b0)], axis=1)

    count = float(B * N)

    h0, stats = pl.pallas_call(
        functools.partial(_stage1_body, cnt_ln=LN),
        grid=(B, N // TN),
        in_specs=[
            pl.BlockSpec((1, 3, TN), lambda b, n: (b, 0, n)),
            pl.BlockSpec((1, M, 3), lambda b, n: (b, 0, 0)),
            pl.BlockSpec((1, C1, TN), lambda b, n: (b, 0, n)),
            pl.BlockSpec((1, C2, M), lambda b, n: (b, 0, 0)),
            pl.BlockSpec((Cout, C1 + C2), lambda b, n: (0, 0)),
            pl.BlockSpec((Cout, 8), lambda b, n: (0, 0)),
        ],
        out_specs=[
            pl.BlockSpec((1, Cout, TN), lambda b, n: (b, 0, n)),
            pl.BlockSpec((2, Cout, LN), lambda b, n: (0, 0, 0)),
        ],
        out_shape=[
            jax.ShapeDtypeStruct((B, Cout, N), f32),
            jax.ShapeDtypeStruct((2, Cout, LN), f32),
        ],
        scratch_shapes=[pltpu.VMEM((M, 128), f32)],
    )(xyz1, x2t, features1, features2, W0, params)

    h1, stats2 = pl.pallas_call(
        functools.partial(_stage2_body, count=count, cnt_ln=LN),
        grid=(B, N // TN2),
        in_specs=[
            pl.BlockSpec((1, Cout, TN2), lambda b, n: (b, 0, n)),
            pl.BlockSpec((2, Cout, LN), lambda b, n: (0, 0, 0)),
            pl.BlockSpec((Cout, Cout), lambda b, n: (0, 0)),
            pl.BlockSpec((Cout, 8), lambda b, n: (0, 0)),
        ],
        out_specs=[
            pl.BlockSpec((1, Cout, TN2), lambda b, n: (b, 0, n)),
            pl.BlockSpec((2, Cout, LN), lambda b, n: (0, 0, 0)),
        ],
        out_shape=[
            jax.ShapeDtypeStruct((B, Cout, N), f32),
            jax.ShapeDtypeStruct((2, Cout, LN), f32),
        ],
    )(h0, stats, W1, params)

    out = pl.pallas_call(
        functools.partial(_stage3_body, count=count),
        grid=(B, N // TN2),
        in_specs=[
            pl.BlockSpec((1, Cout, TN2), lambda b, n: (b, 0, n)),
            pl.BlockSpec((2, Cout, LN), lambda b, n: (0, 0, 0)),
            pl.BlockSpec((Cout, 8), lambda b, n: (0, 0)),
        ],
        out_specs=pl.BlockSpec((1, Cout, TN2), lambda b, n: (b, 0, n)),
        out_shape=jax.ShapeDtypeStruct((B, Cout, N), f32),
    )(h1, stats2, params)

    return out


# speedup vs baseline: 34.7027x; 1.0235x over previous
import functools

import jax
import jax.numpy as jnp
from jax.experimental import pallas as pl
from jax.experimental.pallas import tpu as pltpu

K_NN = 16


def _stage1_body(x1_ref, x2t_ref, f1_ref, f2_ref, w0_ref, p_ref,
                 h0_ref, stats_ref, n2_ref, *, cnt_ln):
    b = pl.program_id(0)
    nt = pl.program_id(1)

    x1 = x1_ref[0]
    x2t = x2t_ref[0]
    f1 = f1_ref[0]
    f2 = f2_ref[0]

    @pl.when(nt == 0)
    def _():
        n2c = (x2t[:, 0:1] * x2t[:, 0:1] + x2t[:, 1:2] * x2t[:, 1:2]
               + x2t[:, 2:3] * x2t[:, 2:3])
        n2_ref[:, 0:1] = n2c

    g = jax.lax.dot_general(x2t.astype(jnp.bfloat16),
                            x1.astype(jnp.bfloat16),
                            (((1,), (0,)), ((), ())),
                            preferred_element_type=jnp.float32)
    n2 = n2_ref[:, 0:1]
    n1 = (x1[0:1, :] * x1[0:1, :] + x1[1:2, :] * x1[1:2, :]
          + x1[2:3, :] * x1[2:3, :])
    d = (-2.0 * g + n1) + n2

    nblk = 8
    bs = d.shape[0] // nblk
    S = [d[i * bs:(i + 1) * bs, :] for i in range(nblk)]
    net = [(0, 1), (2, 3), (4, 5), (6, 7),
           (0, 2), (1, 3), (4, 6), (5, 7),
           (1, 2), (5, 6),
           (0, 4), (1, 5), (2, 6), (3, 7),
           (2, 4), (3, 5),
           (1, 2), (3, 4), (5, 6)]
    for i, j in net:
        lo = jnp.minimum(S[i], S[j])
        hi = jnp.maximum(S[i], S[j])
        S[i], S[j] = lo, hi
    t = None
    for it in range(K_NN):
        t = jnp.min(S[0], axis=0, keepdims=True)
        if it < K_NN - 1:
            pop = S[0] == t
            depth = min(nblk - 1, K_NN - 1 - it)
            for i in range(depth):
                S[i] = jnp.where(pop, S[i + 1], S[i])
            if depth == nblk - 1:
                S[nblk - 1] = jnp.where(pop, jnp.inf, S[nblk - 1])

    mask = d <= t
    r = jnp.where(mask, 1.0 / (d + 1e-8), 0.0)
    norm = jnp.sum(r, axis=0, keepdims=True)

    interp = jax.lax.dot_general(f2.astype(jnp.bfloat16),
                                 r.astype(jnp.bfloat16),
                                 (((1,), (0,)), ((), ())),
                                 preferred_element_type=jnp.float32)
    interp = interp / norm
    nf = jnp.concatenate([f1, interp], axis=0)
    h = jax.lax.dot_general(w0_ref[...].astype(jnp.bfloat16),
                            nf.astype(jnp.bfloat16), (((1,), (0,)), ((), ())),
                            preferred_element_type=jnp.float32)
    h = h + p_ref[:, 0:1]

    h0_ref[0] = h

    tn = h.shape[1]
    hh = h * h
    s = h[:, 0:cnt_ln]
    ss = hh[:, 0:cnt_ln]
    for q in range(1, tn // cnt_ln):
        s = s + h[:, q * cnt_ln:(q + 1) * cnt_ln]
        ss = ss + hh[:, q * cnt_ln:(q + 1) * cnt_ln]

    @pl.when(jnp.logical_and(b == 0, nt == 0))
    def _():
        stats_ref[...] = jnp.zeros_like(stats_ref)

    stats_ref[0] += s
    stats_ref[1] += ss


def _stage2_body(h0_ref, stats_ref, w1_ref, p_ref, h1_ref, stats2_ref,
                 *, count, cnt_ln):
    b = pl.program_id(0)
    nt = pl.program_id(1)

    s = jnp.sum(stats_ref[0], axis=1, keepdims=True)
    ss = jnp.sum(stats_ref[1], axis=1, keepdims=True)
    mean = s / count
    var = ss / count - mean * mean
    rstd = jax.lax.rsqrt(var + 1e-5)

    h0 = h0_ref[0]
    a = (h0 - mean) * rstd * p_ref[:, 1:2] + p_ref[:, 2:3]
    a = jnp.maximum(a, 0.0)
    h1 = jax.lax.dot_general(w1_ref[...].astype(jnp.bfloat16),
                             a.astype(jnp.bfloat16), (((1,), (0,)), ((), ())),
                             preferred_element_type=jnp.float32)
    h1 = h1 + p_ref[:, 3:4]
    h1_ref[0] = h1

    tn = h1.shape[1]
    hh1 = h1 * h1
    s2 = h1[:, 0:cnt_ln]
    ss2 = hh1[:, 0:cnt_ln]
    for q in range(1, tn // cnt_ln):
        s2 = s2 + h1[:, q * cnt_ln:(q + 1) * cnt_ln]
        ss2 = ss2 + hh1[:, q * cnt_ln:(q + 1) * cnt_ln]

    @pl.when(jnp.logical_and(b == 0, nt == 0))
    def _():
        stats2_ref[...] = jnp.zeros_like(stats2_ref)

    stats2_ref[0] += s2
    stats2_ref[1] += ss2


def _stage3_body(h1_ref, stats2_ref, p_ref, out_ref, *, count):
    s = jnp.sum(stats2_ref[0], axis=1, keepdims=True)
    ss = jnp.sum(stats2_ref[1], axis=1, keepdims=True)
    mean = s / count
    var = ss / count - mean * mean
    rstd = jax.lax.rsqrt(var + 1e-5)

    h1 = h1_ref[0]
    y = (h1 - mean) * rstd * p_ref[:, 4:5] + p_ref[:, 5:6]
    out_ref[0] = jnp.maximum(y, 0.0)


def kernel(xyz1, xyz2, features1, features2, W0, b0, g0, be0, W1, b1, g1, be1):
    B, _, N = xyz1.shape
    M = xyz2.shape[2]
    C1 = features1.shape[1]
    C2 = features2.shape[1]
    Cout = W0.shape[0]
    f32 = jnp.float32

    TN = min(1024, N)
    TN2 = min(1024, N)
    LN = 128

    x2t = jnp.transpose(xyz2, (0, 2, 1))
    params = jnp.stack([b0, g0, be0, b1, g1, be1,
                        jnp.zeros_like(b0), jnp.zeros_like(b0)], axis=1)

    count = float(B * N)

    h0, stats = pl.pallas_call(
        functools.partial(_stage1_body, cnt_ln=LN),
        grid=(B, N // TN),
        in_specs=[
            pl.BlockSpec((1, 3, TN), lambda b, n: (b, 0, n)),
            pl.BlockSpec((1, M, 3), lambda b, n: (b, 0, 0)),
            pl.BlockSpec((1, C1, TN), lambda b, n: (b, 0, n)),
            pl.BlockSpec((1, C2, M), lambda b, n: (b, 0, 0)),
            pl.BlockSpec((Cout, C1 + C2), lambda b, n: (0, 0)),
            pl.BlockSpec((Cout, 8), lambda b, n: (0, 0)),
        ],
        out_specs=[
            pl.BlockSpec((1, Cout, TN), lambda b, n: (b, 0, n)),
            pl.BlockSpec((2, Cout, LN), lambda b, n: (0, 0, 0)),
        ],
        out_shape=[
            jax.ShapeDtypeStruct((B, Cout, N), f32),
            jax.ShapeDtypeStruct((2, Cout, LN), f32),
        ],
        scratch_shapes=[pltpu.VMEM((M, 128), f32)],
    )(xyz1, x2t, features1, features2, W0, params)

    h1, stats2 = pl.pallas_call(
        functools.partial(_stage2_body, count=count, cnt_ln=LN),
        grid=(B, N // TN2),
        in_specs=[
            pl.BlockSpec((1, Cout, TN2), lambda b, n: (b, 0, n)),
            pl.BlockSpec((2, Cout, LN), lambda b, n: (0, 0, 0)),
            pl.BlockSpec((Cout, Cout), lambda b, n: (0, 0)),
            pl.BlockSpec((Cout, 8), lambda b, n: (0, 0)),
        ],
        out_specs=[
            pl.BlockSpec((1, Cout, TN2), lambda b, n: (b, 0, n)),
            pl.BlockSpec((2, Cout, LN), lambda b, n: (0, 0, 0)),
        ],
        out_shape=[
            jax.ShapeDtypeStruct((B, Cout, N), f32),
            jax.ShapeDtypeStruct((2, Cout, LN), f32),
        ],
    )(h0, stats, W1, params)

    out = pl.pallas_call(
        functools.partial(_stage3_body, count=count),
        grid=(B, N // TN2),
        in_specs=[
            pl.BlockSpec((1, Cout, TN2), lambda b, n: (b, 0, n)),
            pl.BlockSpec((2, Cout, LN), lambda b, n: (0, 0, 0)),
            pl.BlockSpec((Cout, 8), lambda b, n: (0, 0)),
        ],
        out_specs=pl.BlockSpec((1, Cout, TN2), lambda b, n: (b, 0, n)),
        out_shape=jax.ShapeDtypeStruct((B, Cout, N), f32),
    )(h1, stats2, params)

    return out
